# baseline (speedup 1.0000x reference)
import functools

import jax
import jax.numpy as jnp
from jax import lax
from jax.experimental import pallas as pl
from jax.experimental.pallas import tpu as pltpu
from jax.experimental.pallas import tpu_sc as plsc

LANES = 16
CHUNK = 1280
PITCH = 513


@functools.lru_cache(maxsize=None)
def _build(n_rows: int, vocab: int, channels: int):
    info = plsc.get_sparse_core_info()
    nw = info.num_cores * info.num_subcores
    nc = info.num_cores

    assert n_rows % (nw * CHUNK) == 0 and CHUNK % 128 == 0
    rows_per_w = n_rows // nw
    n_iter = rows_per_w // CHUNK
    assert n_iter % 2 == 0
    groups = CHUNK // (4 * LANES)
    cw = CHUNK * channels
    twords = vocab * channels
    n_words = n_rows // 4
    cwords = CHUNK // 4
    cblks = channels // 8
    iblks = n_rows // 128
    tile_elems = 8 * 128

    mesh = plsc.VectorSubcoreMesh(core_axis_name="c", subcore_axis_name="s")

    @functools.partial(
        pl.kernel,
        out_type=jax.ShapeDtypeStruct((n_rows * channels,), jnp.float32),
        mesh=mesh,
        scratch_types=[
            pltpu.VMEM((LANES * PITCH,), jnp.float32),
            pltpu.VMEM((twords,), jnp.int32),
            pltpu.VMEM((2, cwords), jnp.int32),
            pltpu.VMEM((2, cw), jnp.float32),
            pltpu.SemaphoreType.DMA,
            pltpu.SemaphoreType.DMA,
            pltpu.SemaphoreType.DMA,
            pltpu.SemaphoreType.DMA,
        ],
        compiler_params=pltpu.CompilerParams(
            use_tc_tiling_on_sc=False, needs_layout_passes=False
        ),
    )
    def lookup(
        data_hbm, out_hbm, table_v, table_v8, idx_v, buf_v, so0, so1, si0, si1
    ):
        wid = lax.axis_index("s") * nc + lax.axis_index("c")
        row0 = wid * rows_per_w
        row0w = wid * (rows_per_w // 4)
        out_sems = (so0, so1)
        idx_sems = (si0, si1)

        pltpu.sync_copy(data_hbm.at[pl.ds(n_words, twords)], table_v8)

        @pl.loop(0, LANES)
        def _(t):
            for ws in range(twords // LANES):
                table_v[pl.ds(t * PITCH + ws * LANES, LANES)] = plsc.bitcast(
                    table_v8[pl.ds(ws * LANES, LANES)], jnp.float32
                )

        lane513 = lax.iota(jnp.int32, LANES) * PITCH

        pltpu.async_copy(
            data_hbm.at[pl.ds(row0w, cwords)], idx_v.at[0], idx_sems[0]
        )

        @pl.loop(0, n_iter, step=2)
        def _(it):
            for b in range(2):
                i = it + b
                start = row0 + i * CHUNK
                startw = pl.multiple_of(row0w + i * cwords, cwords)
                iblk0 = pl.multiple_of(
                    wid * (rows_per_w // 128) + i * (CHUNK // 128),
                    CHUNK // 128,
                )
                bufb = buf_v.at[b]
                idxb = idx_v.at[b]

                pltpu.make_async_copy(
                    data_hbm.at[pl.ds(startw, cwords)], idxb, idx_sems[b]
                ).wait()

                @pl.when(i + 1 < n_iter)
                def _prefetch():
                    pltpu.async_copy(
                        data_hbm.at[pl.ds(startw + cwords, cwords)],
                        idx_v.at[1 - b],
                        idx_sems[1 - b],
                    )

                @pl.when(i >= 2)
                def _drain():
                    for cb in range(cblks):
                        pltpu.make_async_copy(
                            out_hbm.at[pl.ds(0, CHUNK * 8)],
                            bufb.at[pl.ds(cb * CHUNK * 8, CHUNK * 8)],
                            out_sems[b],
                        ).wait()

                @pl.loop(0, groups)
                def _(g):
                    w = idxb[pl.ds(g * LANES, LANES)]
                    for j in range(4):
                        gbase = lane513 + (
                            (lax.shift_right_logical(w, 8 * j) & 0xFF)
                            * channels
                        )
                        s = g * 4 + j
                        soff = (
                            lax.shift_left(
                                lax.shift_right_logical(s, 3), 10
                            )
                            + lax.shift_left(s & 7, 4)
                        )
                        for ch in range(channels):
                            f_ch = (ch // 8) * (CHUNK * 8) + (ch % 8) * 128
                            vals = plsc.load_gather(table_v, [gbase + ch])
                            bufb[pl.ds(soff + f_ch, LANES)] = vals

                for cb in range(cblks):
                    pltpu.async_copy(
                        bufb.at[pl.ds(cb * CHUNK * 8, CHUNK * 8)],
                        out_hbm.at[
                            pl.ds((cb * iblks + iblk0) * tile_elems, CHUNK * 8)
                        ],
                        out_sems[b],
                    )

        for b in range(2):
            for cb in range(cblks):
                pltpu.make_async_copy(
                    out_hbm.at[pl.ds(0, CHUNK * 8)],
                    buf_v.at[b].at[pl.ds(cb * CHUNK * 8, CHUNK * 8)],
                    out_sems[b],
                ).wait()

    return lookup


def kernel(prior, emb):
    n_rows = prior.size
    vocab, channels = emb.shape
    blk = prior.reshape(n_rows // 64, 4, 16)
    w = (
        blk[:, 0]
        + (blk[:, 1] << 8)
        + (blk[:, 2] << 16)
        + (blk[:, 3] << 24)
    ).reshape(n_rows // 4)
    embw = lax.bitcast_convert_type(emb, jnp.int32).reshape(-1)
    data = jnp.concatenate([w, embw])
    flat = _build(n_rows, vocab, channels)(data)
    out4 = flat.reshape(channels // 8, n_rows // 128, 8, 128)
    return out4.transpose(1, 3, 0, 2).reshape(n_rows, channels)


# speedup vs baseline: 1.0013x; 1.0013x over previous
import functools

import jax
import jax.numpy as jnp
from jax import lax
from jax.experimental import pallas as pl
from jax.experimental.pallas import tpu as pltpu
from jax.experimental.pallas import tpu_sc as plsc

LANES = 16
CHUNK = 1024
PITCH = 513


@functools.lru_cache(maxsize=None)
def _build(n_rows: int, vocab: int, channels: int):
    info = plsc.get_sparse_core_info()
    nw = info.num_cores * info.num_subcores
    nc = info.num_cores

    assert n_rows % (nw * CHUNK) == 0 and CHUNK % 128 == 0
    rows_per_w = n_rows // nw
    n_iter = rows_per_w // CHUNK
    assert n_iter % 2 == 0
    groups = CHUNK // (4 * LANES)
    cw = CHUNK * channels
    twords = vocab * channels
    n_words = n_rows // 4
    cwords = CHUNK // 4
    cblks = channels // 8
    iblks = n_rows // 128
    tile_elems = 8 * 128

    mesh = plsc.VectorSubcoreMesh(core_axis_name="c", subcore_axis_name="s")

    @functools.partial(
        pl.kernel,
        out_type=jax.ShapeDtypeStruct((n_rows * channels,), jnp.float32),
        mesh=mesh,
        scratch_types=[
            pltpu.VMEM((LANES * PITCH,), jnp.float32),
            pltpu.VMEM((twords,), jnp.int32),
            pltpu.VMEM((2, cwords), jnp.int32),
            pltpu.VMEM((2, cw), jnp.float32),
            pltpu.SemaphoreType.DMA,
            pltpu.SemaphoreType.DMA,
            pltpu.SemaphoreType.DMA,
            pltpu.SemaphoreType.DMA,
        ],
        compiler_params=pltpu.CompilerParams(
            use_tc_tiling_on_sc=False, needs_layout_passes=False
        ),
    )
    def lookup(
        data_hbm, out_hbm, table_v, table_v8, idx_v, buf_v, so0, so1, si0, si1
    ):
        wid = lax.axis_index("s") * nc + lax.axis_index("c")
        row0 = wid * rows_per_w
        row0w = wid * (rows_per_w // 4)
        out_sems = (so0, so1)
        idx_sems = (si0, si1)

        pltpu.sync_copy(data_hbm.at[pl.ds(n_words, twords)], table_v8)

        @pl.loop(0, LANES)
        def _(t):
            for ws in range(twords // LANES):
                table_v[pl.ds(t * PITCH + ws * LANES, LANES)] = plsc.bitcast(
                    table_v8[pl.ds(ws * LANES, LANES)], jnp.float32
                )

        lane513 = lax.iota(jnp.int32, LANES) * PITCH

        pltpu.async_copy(
            data_hbm.at[pl.ds(row0w, cwords)], idx_v.at[0], idx_sems[0]
        )

        @pl.loop(0, n_iter, step=2)
        def _(it):
            for b in range(2):
                i = it + b
                start = row0 + i * CHUNK
                startw = pl.multiple_of(row0w + i * cwords, cwords)
                iblk0 = pl.multiple_of(
                    wid * (rows_per_w // 128) + i * (CHUNK // 128),
                    CHUNK // 128,
                )
                bufb = buf_v.at[b]
                idxb = idx_v.at[b]

                pltpu.make_async_copy(
                    data_hbm.at[pl.ds(startw, cwords)], idxb, idx_sems[b]
                ).wait()

                @pl.when(i + 1 < n_iter)
                def _prefetch():
                    pltpu.async_copy(
                        data_hbm.at[pl.ds(startw + cwords, cwords)],
                        idx_v.at[1 - b],
                        idx_sems[1 - b],
                    )

                @pl.when(i >= 2)
                def _drain():
                    for cb in range(cblks):
                        pltpu.make_async_copy(
                            out_hbm.at[pl.ds(0, CHUNK * 8)],
                            bufb.at[pl.ds(cb * CHUNK * 8, CHUNK * 8)],
                            out_sems[b],
                        ).wait()

                @pl.loop(0, groups)
                def _(g):
                    w = idxb[pl.ds(g * LANES, LANES)]
                    for j in range(4):
                        gbase = lane513 + (
                            (lax.shift_right_logical(w, 8 * j) & 0xFF)
                            * channels
                        )
                        s = g * 4 + j
                        soff = (
                            lax.shift_left(
                                lax.shift_right_logical(s, 3), 10
                            )
                            + lax.shift_left(s & 7, 4)
                        )
                        for ch in range(channels):
                            f_ch = (ch // 8) * (CHUNK * 8) + (ch % 8) * 128
                            vals = plsc.load_gather(table_v, [gbase + ch])
                            bufb[pl.ds(soff + f_ch, LANES)] = vals

                for cb in range(cblks):
                    pltpu.async_copy(
                        bufb.at[pl.ds(cb * CHUNK * 8, CHUNK * 8)],
                        out_hbm.at[
                            pl.ds((cb * iblks + iblk0) * tile_elems, CHUNK * 8)
                        ],
                        out_sems[b],
                    )

        for b in range(2):
            for cb in range(cblks):
                pltpu.make_async_copy(
                    out_hbm.at[pl.ds(0, CHUNK * 8)],
                    buf_v.at[b].at[pl.ds(cb * CHUNK * 8, CHUNK * 8)],
                    out_sems[b],
                ).wait()

    return lookup


def kernel(prior, emb):
    n_rows = prior.size
    vocab, channels = emb.shape
    blk = prior.reshape(n_rows // 64, 4, 16)
    w = (
        blk[:, 0]
        + (blk[:, 1] << 8)
        + (blk[:, 2] << 16)
        + (blk[:, 3] << 24)
    ).reshape(n_rows // 4)
    embw = lax.bitcast_convert_type(emb, jnp.int32).reshape(-1)
    data = jnp.concatenate([w, embw])
    flat = _build(n_rows, vocab, channels)(data)
    out4 = flat.reshape(channels // 8, n_rows // 128, 8, 128)
    return out4.transpose(1, 3, 0, 2).reshape(n_rows, channels)
